# baseline (speedup 1.0000x reference)
import jax
import jax.numpy as jnp
from jax.experimental import pallas as pl
from jax.experimental.pallas import tpu as pltpu

NUM_THETAS = 32
BUMP_STEPS = 32
NUM_FEATURES = 128
R = 1.1
SCALE = 8.0
NG = 64
BLK = 2048
ST = BUMP_STEPS * NUM_THETAS


def _fused(batch_ref, x_ref, v_ref, k_ref, out_ref, cnt_ref):
    i = pl.program_id(1)

    @pl.when(i == 0)
    def _init():
        out_ref[...] = jnp.zeros_like(out_ref)
        cnt_ref[...] = jnp.zeros_like(cnt_ref)

    x = x_ref[...]
    v = v_ref[...]
    nh = jax.lax.dot_general(
        x, v, (((1,), (1,)), ((), ())), preferred_element_type=jnp.float32
    )
    p = jnp.exp(SCALE * nh)
    p_t = jnp.concatenate([p] * BUMP_STEPS, axis=1)
    ecc = 1.0 / (1.0 + k_ref[...] * p_t)

    b = batch_ref[0, 0, :]
    g = jax.lax.broadcasted_iota(jnp.int32, (NG, BLK), 0)
    onehot = (g == b[None, :]).astype(jnp.bfloat16)
    contrib = jax.lax.dot_general(
        onehot, ecc.astype(jnp.bfloat16),
        (((1,), (0,)), ((), ())), preferred_element_type=jnp.float32,
    )
    out_ref[0] += contrib
    ones = jnp.ones((BLK, 128), dtype=jnp.bfloat16)
    cnt = jax.lax.dot_general(
        onehot, ones, (((1,), (0,)), ((), ())),
        preferred_element_type=jnp.float32,
    )
    cnt_ref[0] += cnt


NCORES = 2


def kernel(x, batch, num_graphs, v):
    del num_graphs
    n = x.shape[0]
    per_core = (n + NCORES * BLK - 1) // (NCORES * BLK)
    nblocks = per_core * NCORES
    npad = nblocks * BLK - n
    if npad:
        x = jnp.pad(x, ((0, npad), (0, 0)))
        batch = jnp.pad(batch, (0, npad), constant_values=NG)
    batch3 = batch.reshape(nblocks, 1, BLK)

    lin = jnp.linspace(-R, R, BUMP_STEPS, dtype=jnp.float32)
    k = jnp.repeat(jnp.exp(-SCALE * lin), NUM_THETAS).reshape(1, ST)
    c = jnp.repeat(jax.nn.sigmoid(SCALE * (lin - R)), NUM_THETAS).reshape(1, ST)

    out, cnt = pl.pallas_call(
        _fused,
        grid=(NCORES, per_core),
        in_specs=[
            pl.BlockSpec((1, 1, BLK), lambda c_, i: (c_ * per_core + i, 0, 0)),
            pl.BlockSpec((BLK, NUM_FEATURES), lambda c_, i: (c_ * per_core + i, 0)),
            pl.BlockSpec((NUM_THETAS, NUM_FEATURES), lambda c_, i: (0, 0)),
            pl.BlockSpec((1, ST), lambda c_, i: (0, 0)),
        ],
        out_specs=[
            pl.BlockSpec((1, NG, ST), lambda c_, i: (c_, 0, 0)),
            pl.BlockSpec((1, NG, 128), lambda c_, i: (c_, 0, 0)),
        ],
        out_shape=[
            jax.ShapeDtypeStruct((NCORES, NG, ST), jnp.float32),
            jax.ShapeDtypeStruct((NCORES, NG, 128), jnp.float32),
        ],
        compiler_params=pltpu.CompilerParams(
            dimension_semantics=("parallel", "arbitrary"),
        ),
    )(batch3, x, v, k)
    out = out.sum(axis=0) - cnt.sum(axis=0)[:, :1] * c
    return out.reshape(NG, BUMP_STEPS, NUM_THETAS)


# speedup vs baseline: 1.2366x; 1.2366x over previous
import jax
import jax.numpy as jnp
import numpy as np
from jax.experimental import pallas as pl
from jax.experimental.pallas import tpu as pltpu

NUM_THETAS = 32
BUMP_STEPS = 32
NUM_FEATURES = 128
R = 1.1
SCALE = 8.0
NG = 64
BLK = 2048
ST = BUMP_STEPS * NUM_THETAS

_LIN = np.linspace(-R, R, BUMP_STEPS, dtype=np.float32)
_E = [float(v) for v in np.exp(np.float64(SCALE) * _LIN)]


def _fused(batch_ref, x_ref, vs_ref, out_ref, cnt_ref):
    i = pl.program_id(0)

    @pl.when(i == 0)
    def _init():
        out_ref[...] = jnp.zeros_like(out_ref)
        cnt_ref[...] = jnp.zeros_like(cnt_ref)

    x = x_ref[...]
    vs = vs_ref[...]
    nh = jax.lax.dot_general(
        vs, x, (((1,), (1,)), ((), ())), preferred_element_type=jnp.float32
    )
    p = jnp.exp(nh)
    q = jnp.concatenate(
        [(1.0 / (p + _E[s])).astype(jnp.bfloat16) for s in range(BUMP_STEPS)],
        axis=0,
    )

    b = batch_ref[0, 0, :]
    g = jax.lax.broadcasted_iota(jnp.int32, (NG, BLK), 0)
    onehot = (g == b[None, :]).astype(jnp.bfloat16)
    contrib = jax.lax.dot_general(
        q, onehot, (((1,), (1,)), ((), ())), preferred_element_type=jnp.float32,
    )
    out_ref[...] += contrib
    ones = jnp.ones((BLK, 128), dtype=jnp.bfloat16)
    cnt = jax.lax.dot_general(
        onehot, ones, (((1,), (0,)), ((), ())),
        preferred_element_type=jnp.float32,
    )
    cnt_ref[...] += cnt


def kernel(x, batch, num_graphs, v):
    del num_graphs
    n = x.shape[0]
    nblocks = (n + BLK - 1) // BLK
    npad = nblocks * BLK - n
    if npad:
        x = jnp.pad(x, ((0, npad), (0, 0)))
        batch = jnp.pad(batch, (0, npad), constant_values=NG)
    batch3 = batch.reshape(nblocks, 1, BLK)
    vs = SCALE * v

    out, cnt = pl.pallas_call(
        _fused,
        grid=(nblocks,),
        in_specs=[
            pl.BlockSpec((1, 1, BLK), lambda i: (i, 0, 0)),
            pl.BlockSpec((BLK, NUM_FEATURES), lambda i: (i, 0)),
            pl.BlockSpec((NUM_THETAS, NUM_FEATURES), lambda i: (0, 0)),
        ],
        out_specs=[
            pl.BlockSpec((ST, NG), lambda i: (0, 0)),
            pl.BlockSpec((NG, 128), lambda i: (0, 0)),
        ],
        out_shape=[
            jax.ShapeDtypeStruct((ST, NG), jnp.float32),
            jax.ShapeDtypeStruct((NG, 128), jnp.float32),
        ],
        compiler_params=pltpu.CompilerParams(
            dimension_semantics=("arbitrary",),
        ),
    )(batch3, x, vs)

    lin = jnp.asarray(_LIN)
    e_col = jnp.repeat(jnp.exp(SCALE * lin), NUM_THETAS)[:, None]
    c_row = jnp.repeat(jax.nn.sigmoid(SCALE * (lin - R)), NUM_THETAS)[None, :]
    res = (out * e_col).T - cnt[:, :1] * c_row
    return res.reshape(NG, BUMP_STEPS, NUM_THETAS)


# speedup vs baseline: 1.2894x; 1.0426x over previous
import jax
import jax.numpy as jnp
import numpy as np
from jax.experimental import pallas as pl
from jax.experimental.pallas import tpu as pltpu

NUM_THETAS = 32
BUMP_STEPS = 32
NUM_FEATURES = 128
R = 1.1
SCALE = 8.0
NG = 64
BLK = 2048
ST = BUMP_STEPS * NUM_THETAS

_LIN = np.linspace(-R, R, BUMP_STEPS, dtype=np.float32)
_E = [float(v) for v in np.exp(np.float64(SCALE) * _LIN)]


def _fused(batch_ref, x_ref, vs_ref, out_ref, cnt_ref):
    i = pl.program_id(0)

    @pl.when(i == 0)
    def _init():
        out_ref[...] = jnp.zeros_like(out_ref)
        cnt_ref[...] = jnp.zeros_like(cnt_ref)

    x = x_ref[...]
    vs = vs_ref[...]
    nh = jax.lax.dot_general(
        vs, x, (((1,), (1,)), ((), ())), preferred_element_type=jnp.float32
    )
    p = jnp.exp(nh)
    q = jnp.concatenate(
        [(1.0 / (p + _E[s])).astype(jnp.bfloat16) for s in range(BUMP_STEPS)],
        axis=0,
    )

    b = batch_ref[0, 0, :]
    g = jax.lax.broadcasted_iota(jnp.int32, (NG, BLK), 0)
    onehot = (g == b[None, :]).astype(jnp.bfloat16)
    contrib = jax.lax.dot_general(
        q, onehot, (((1,), (1,)), ((), ())), preferred_element_type=jnp.float32,
    )
    out_ref[...] += contrib
    ones = jnp.ones((BLK, 128), dtype=jnp.bfloat16)
    cnt = jax.lax.dot_general(
        onehot, ones, (((1,), (0,)), ((), ())),
        preferred_element_type=jnp.float32,
    )
    cnt_ref[...] += cnt


def kernel(x, batch, num_graphs, v):
    del num_graphs
    n = x.shape[0]
    nblocks = (n + BLK - 1) // BLK
    npad = nblocks * BLK - n
    if npad:
        x = jnp.pad(x, ((0, npad), (0, 0)))
        batch = jnp.pad(batch, (0, npad), constant_values=NG)
    batch3 = batch.reshape(nblocks, 1, BLK)
    x = x.astype(jnp.bfloat16)
    vs = (SCALE * v).astype(jnp.bfloat16)

    out, cnt = pl.pallas_call(
        _fused,
        grid=(nblocks,),
        in_specs=[
            pl.BlockSpec((1, 1, BLK), lambda i: (i, 0, 0)),
            pl.BlockSpec((BLK, NUM_FEATURES), lambda i: (i, 0)),
            pl.BlockSpec((NUM_THETAS, NUM_FEATURES), lambda i: (0, 0)),
        ],
        out_specs=[
            pl.BlockSpec((ST, NG), lambda i: (0, 0)),
            pl.BlockSpec((NG, 128), lambda i: (0, 0)),
        ],
        out_shape=[
            jax.ShapeDtypeStruct((ST, NG), jnp.float32),
            jax.ShapeDtypeStruct((NG, 128), jnp.float32),
        ],
        compiler_params=pltpu.CompilerParams(
            dimension_semantics=("arbitrary",),
        ),
    )(batch3, x, vs)

    lin = jnp.asarray(_LIN)
    e_col = jnp.repeat(jnp.exp(SCALE * lin), NUM_THETAS)[:, None]
    c_row = jnp.repeat(jax.nn.sigmoid(SCALE * (lin - R)), NUM_THETAS)[None, :]
    res = (out * e_col).T - cnt[:, :1] * c_row
    return res.reshape(NG, BUMP_STEPS, NUM_THETAS)


# speedup vs baseline: 1.2954x; 1.0047x over previous
import jax
import jax.numpy as jnp
import numpy as np
from jax.experimental import pallas as pl
from jax.experimental.pallas import tpu as pltpu

NUM_THETAS = 32
BUMP_STEPS = 32
NUM_FEATURES = 128
R = 1.1
SCALE = 8.0
NG = 64
BLK = 2048
ST = BUMP_STEPS * NUM_THETAS

_LIN = np.linspace(-R, R, BUMP_STEPS, dtype=np.float32)
_E = [float(v) for v in np.exp(np.float64(SCALE) * _LIN)]


def _fused(batch_ref, x_ref, vs_ref, out_ref, cnt_ref):
    i = pl.program_id(0)

    @pl.when(i == 0)
    def _init():
        out_ref[...] = jnp.zeros_like(out_ref)
        cnt_ref[...] = jnp.zeros_like(cnt_ref)

    x = x_ref[...]
    vs = vs_ref[...]
    nh = jax.lax.dot_general(
        vs, x, (((1,), (1,)), ((), ())), preferred_element_type=jnp.float32
    )
    p = jnp.exp(nh).astype(jnp.bfloat16)
    q = jnp.concatenate(
        [1.0 / (p + jnp.bfloat16(_E[s])) for s in range(BUMP_STEPS)],
        axis=0,
    )

    b = batch_ref[0, 0, :]
    g = jax.lax.broadcasted_iota(jnp.int32, (NG, BLK), 0)
    onehot = (g == b[None, :]).astype(jnp.bfloat16)
    contrib = jax.lax.dot_general(
        q, onehot, (((1,), (1,)), ((), ())), preferred_element_type=jnp.float32,
    )
    out_ref[...] += contrib
    ones = jnp.ones((BLK, 128), dtype=jnp.bfloat16)
    cnt = jax.lax.dot_general(
        onehot, ones, (((1,), (0,)), ((), ())),
        preferred_element_type=jnp.float32,
    )
    cnt_ref[...] += cnt


def kernel(x, batch, num_graphs, v):
    del num_graphs
    n = x.shape[0]
    nblocks = (n + BLK - 1) // BLK
    npad = nblocks * BLK - n
    if npad:
        x = jnp.pad(x, ((0, npad), (0, 0)))
        batch = jnp.pad(batch, (0, npad), constant_values=NG)
    batch3 = batch.reshape(nblocks, 1, BLK)
    x = x.astype(jnp.bfloat16)
    vs = (SCALE * v).astype(jnp.bfloat16)

    out, cnt = pl.pallas_call(
        _fused,
        grid=(nblocks,),
        in_specs=[
            pl.BlockSpec((1, 1, BLK), lambda i: (i, 0, 0)),
            pl.BlockSpec((BLK, NUM_FEATURES), lambda i: (i, 0)),
            pl.BlockSpec((NUM_THETAS, NUM_FEATURES), lambda i: (0, 0)),
        ],
        out_specs=[
            pl.BlockSpec((ST, NG), lambda i: (0, 0)),
            pl.BlockSpec((NG, 128), lambda i: (0, 0)),
        ],
        out_shape=[
            jax.ShapeDtypeStruct((ST, NG), jnp.float32),
            jax.ShapeDtypeStruct((NG, 128), jnp.float32),
        ],
        compiler_params=pltpu.CompilerParams(
            dimension_semantics=("arbitrary",),
        ),
    )(batch3, x, vs)

    lin = jnp.asarray(_LIN)
    e_col = jnp.repeat(jnp.exp(SCALE * lin), NUM_THETAS)[:, None]
    c_row = jnp.repeat(jax.nn.sigmoid(SCALE * (lin - R)), NUM_THETAS)[None, :]
    res = (out * e_col).T - cnt[:, :1] * c_row
    return res.reshape(NG, BUMP_STEPS, NUM_THETAS)
